# baseline (device time: 97517 ns/iter reference)
import jax
import jax.numpy as jnp
from jax import lax
from jax.experimental import pallas as pl
from jax.experimental.pallas import tpu as pltpu

N_DEV = 4


def kernel(x, W1, W2):
    m_per, d = x.shape

    def body(x_ref, w1_ref, w2_ref, out_ref,
             xr_ref, p_ref, rs_ref, ag_send, ag_recv, rs_send, rs_recv):
        my = lax.axis_index("i")
        left = lax.rem(my + N_DEV - 1, N_DEV)
        right = lax.rem(my + 1, N_DEV)

        barrier = pltpu.get_barrier_semaphore()
        for nbr in (left, right):
            pl.semaphore_signal(barrier, inc=1, device_id=(nbr,),
                                device_id_type=pl.DeviceIdType.MESH)
        pl.semaphore_wait(barrier, 2)

        xr_ref[0] = x_ref[:, :]

        for h in range(N_DEV - 1):
            rdma = pltpu.make_async_remote_copy(
                src_ref=xr_ref.at[h],
                dst_ref=xr_ref.at[h + 1],
                send_sem=ag_send.at[h],
                recv_sem=ag_recv.at[h],
                device_id=(right,),
                device_id_type=pl.DeviceIdType.MESH,
            )
            rdma.start()
            rdma.wait()

        for j in range(N_DEV):
            xc = xr_ref[(j + 1) % N_DEV]
            h1 = jnp.dot(xc, w1_ref[:, :], preferred_element_type=jnp.float32)
            h1 = h1 * (1.0 / (1.0 + jnp.exp(-h1)))
            p_ref[j] = jnp.dot(h1, w2_ref[:, :],
                               preferred_element_type=jnp.float32)

        for s in range(N_DEV - 1):
            rdma = pltpu.make_async_remote_copy(
                src_ref=p_ref.at[s],
                dst_ref=rs_ref.at[s],
                send_sem=rs_send.at[s],
                recv_sem=rs_recv.at[s],
                device_id=(right,),
                device_id_type=pl.DeviceIdType.MESH,
            )
            rdma.start()
            rdma.wait()
            p_ref[s + 1] = p_ref[s + 1] + rs_ref[s]

        out_ref[:, :] = p_ref[N_DEV - 1]

    return pl.pallas_call(
        body,
        out_shape=jax.ShapeDtypeStruct((m_per, d), jnp.float32),
        in_specs=[pl.BlockSpec(memory_space=pltpu.VMEM)] * 3,
        out_specs=pl.BlockSpec(memory_space=pltpu.VMEM),
        scratch_shapes=[
            pltpu.VMEM((N_DEV, m_per, d), jnp.float32),
            pltpu.VMEM((N_DEV, m_per, d), jnp.float32),
            pltpu.VMEM((N_DEV - 1, m_per, d), jnp.float32),
            pltpu.SemaphoreType.DMA((N_DEV - 1,)),
            pltpu.SemaphoreType.DMA((N_DEV - 1,)),
            pltpu.SemaphoreType.DMA((N_DEV - 1,)),
            pltpu.SemaphoreType.DMA((N_DEV - 1,)),
        ],
        compiler_params=pltpu.CompilerParams(collective_id=0),
    )(x, W1, W2)


# device time: 50363 ns/iter; 1.9363x vs baseline; 1.9363x over previous
import jax
import jax.numpy as jnp
from jax import lax
from jax.experimental import pallas as pl
from jax.experimental.pallas import tpu as pltpu

N_DEV = 4


def kernel(x, W1, W2):
    m_per, d = x.shape
    mh = m_per // 2

    def body(x_ref, w1_ref, w2_ref, out_ref,
             xrA, pA, rsA, xrB, pB, rsB,
             agA_s, agA_r, rsA_s, rsA_r,
             agB_s, agB_r, rsB_s, rsB_r):
        my = lax.axis_index("i")
        left = lax.rem(my + N_DEV - 1, N_DEV)
        right = lax.rem(my + 1, N_DEV)

        barrier = pltpu.get_barrier_semaphore()
        for nbr in (left, right):
            pl.semaphore_signal(barrier, inc=1, device_id=(nbr,),
                                device_id_type=pl.DeviceIdType.MESH)
        pl.semaphore_wait(barrier, 2)

        def ag_rdma(xr, ssem, rsem, h, dst):
            return pltpu.make_async_remote_copy(
                src_ref=xr.at[h], dst_ref=xr.at[h + 1],
                send_sem=ssem.at[h], recv_sem=rsem.at[h],
                device_id=(dst,), device_id_type=pl.DeviceIdType.MESH)

        def rs_rdma(p, rs, ssem, rsem, s, dst):
            return pltpu.make_async_remote_copy(
                src_ref=p.at[s], dst_ref=rs.at[s],
                send_sem=ssem.at[s], recv_sem=rsem.at[s],
                device_id=(dst,), device_id_type=pl.DeviceIdType.MESH)

        def f(xr, j):
            h1 = jnp.dot(xr[j], w1_ref[:, :],
                         preferred_element_type=jnp.float32)
            h1 = h1 * (1.0 / (1.0 + jnp.exp(-h1)))
            return jnp.dot(h1, w2_ref[:, :],
                           preferred_element_type=jnp.float32)

        sends = []

        def start(desc):
            desc.start()
            sends.append(desc)
            return desc

        xrA[0] = x_ref[:mh, :]
        xrB[0] = x_ref[mh:, :]

        a0 = start(ag_rdma(xrA, agA_s, agA_r, 0, right))
        b0 = start(ag_rdma(xrB, agB_s, agB_r, 0, left))

        pA[3] = f(xrA, 0)
        pB[3] = f(xrB, 0)

        a0.wait_recv()
        a1 = start(ag_rdma(xrA, agA_s, agA_r, 1, right))
        pA[0] = f(xrA, 1)
        ra0 = start(rs_rdma(pA, rsA, rsA_s, rsA_r, 0, right))

        b0.wait_recv()
        b1 = start(ag_rdma(xrB, agB_s, agB_r, 1, left))
        pB[0] = f(xrB, 1)
        rb0 = start(rs_rdma(pB, rsB, rsB_s, rsB_r, 0, left))

        a1.wait_recv()
        a2 = start(ag_rdma(xrA, agA_s, agA_r, 2, right))
        pA[1] = f(xrA, 2)
        b1.wait_recv()
        b2 = start(ag_rdma(xrB, agB_s, agB_r, 2, left))
        pB[1] = f(xrB, 2)

        ra0.wait_recv()
        pA[1] = pA[1] + rsA[0]
        ra1 = start(rs_rdma(pA, rsA, rsA_s, rsA_r, 1, right))
        rb0.wait_recv()
        pB[1] = pB[1] + rsB[0]
        rb1 = start(rs_rdma(pB, rsB, rsB_s, rsB_r, 1, left))

        a2.wait_recv()
        pA[2] = f(xrA, 3)
        b2.wait_recv()
        pB[2] = f(xrB, 3)

        ra1.wait_recv()
        pA[2] = pA[2] + rsA[1]
        ra2 = start(rs_rdma(pA, rsA, rsA_s, rsA_r, 2, right))
        rb1.wait_recv()
        pB[2] = pB[2] + rsB[1]
        rb2 = start(rs_rdma(pB, rsB, rsB_s, rsB_r, 2, left))

        ra2.wait_recv()
        out_ref[:mh, :] = pA[3] + rsA[2]
        rb2.wait_recv()
        out_ref[mh:, :] = pB[3] + rsB[2]

        for desc in sends:
            desc.wait_send()

    half = (N_DEV, mh, d)
    return pl.pallas_call(
        body,
        out_shape=jax.ShapeDtypeStruct((m_per, d), jnp.float32),
        in_specs=[pl.BlockSpec(memory_space=pltpu.VMEM)] * 3,
        out_specs=pl.BlockSpec(memory_space=pltpu.VMEM),
        scratch_shapes=[
            pltpu.VMEM(half, jnp.float32),
            pltpu.VMEM(half, jnp.float32),
            pltpu.VMEM((N_DEV - 1, mh, d), jnp.float32),
            pltpu.VMEM(half, jnp.float32),
            pltpu.VMEM(half, jnp.float32),
            pltpu.VMEM((N_DEV - 1, mh, d), jnp.float32),
            pltpu.SemaphoreType.DMA((N_DEV - 1,)),
            pltpu.SemaphoreType.DMA((N_DEV - 1,)),
            pltpu.SemaphoreType.DMA((N_DEV - 1,)),
            pltpu.SemaphoreType.DMA((N_DEV - 1,)),
            pltpu.SemaphoreType.DMA((N_DEV - 1,)),
            pltpu.SemaphoreType.DMA((N_DEV - 1,)),
            pltpu.SemaphoreType.DMA((N_DEV - 1,)),
            pltpu.SemaphoreType.DMA((N_DEV - 1,)),
        ],
        compiler_params=pltpu.CompilerParams(collective_id=0),
    )(x, W1, W2)


# device time: 44637 ns/iter; 2.1847x vs baseline; 1.1283x over previous
import jax
import jax.numpy as jnp
from jax import lax
from jax.experimental import pallas as pl
from jax.experimental.pallas import tpu as pltpu

N_DEV = 4
SEG = 2


def kernel(x, W1, W2):
    m_per, d = x.shape
    mh = m_per // 2
    sr = mh // SEG

    def body(x_ref, w1_ref, w2_ref, out_ref,
             xrA, pA, rsA, xrB, pB, rsB,
             agA_s, agA_r, rsA_s, rsA_r,
             agB_s, agB_r, rsB_s, rsB_r):
        my = lax.axis_index("i")
        left = lax.rem(my + N_DEV - 1, N_DEV)
        right = lax.rem(my + 1, N_DEV)

        barrier = pltpu.get_barrier_semaphore()
        for nbr in (left, right):
            pl.semaphore_signal(barrier, inc=1, device_id=(nbr,),
                                device_id_type=pl.DeviceIdType.MESH)
        pl.semaphore_wait(barrier, 2)

        rings = (
            dict(xr=xrA, p=pA, rs=rsA, ag_s=agA_s, ag_r=agA_r,
                 rs_s=rsA_s, rs_r=rsA_r, dst=right, off=0),
            dict(xr=xrB, p=pB, rs=rsB, ag_s=agB_s, ag_r=agB_r,
                 rs_s=rsB_s, rs_r=rsB_r, dst=left, off=mh),
        )

        def rows(g):
            return slice(g * sr, (g + 1) * sr)

        def ag_rdma(r, h, g):
            return pltpu.make_async_remote_copy(
                src_ref=r["xr"].at[h, rows(g), :],
                dst_ref=r["xr"].at[h + 1, rows(g), :],
                send_sem=r["ag_s"].at[h, g],
                recv_sem=r["ag_r"].at[h, g],
                device_id=(r["dst"],), device_id_type=pl.DeviceIdType.MESH)

        def rs_rdma(r, s, g):
            return pltpu.make_async_remote_copy(
                src_ref=r["p"].at[s, rows(g), :],
                dst_ref=r["rs"].at[s, rows(g), :],
                send_sem=r["rs_s"].at[s, g],
                recv_sem=r["rs_r"].at[s, g],
                device_id=(r["dst"],), device_id_type=pl.DeviceIdType.MESH)

        def f(r, j, g):
            xc = r["xr"][j, rows(g), :]
            h1 = jnp.dot(xc, w1_ref[:, :],
                         preferred_element_type=jnp.float32)
            h1 = h1 * (1.0 / (1.0 + jnp.exp(-h1)))
            return jnp.dot(h1, w2_ref[:, :],
                           preferred_element_type=jnp.float32)

        sends = []

        def start(desc):
            desc.start()
            sends.append(desc)
            return desc

        ag = {}
        rs = {}

        for ri, r in enumerate(rings):
            r["xr"][0] = x_ref[r["off"]:r["off"] + mh, :]
        for g in range(SEG):
            for ri, r in enumerate(rings):
                ag[ri, 0, g] = start(ag_rdma(r, 0, g))
        for g in range(SEG):
            for ri, r in enumerate(rings):
                r["p"][3, rows(g), :] = f(r, 0, g)

        for g in range(SEG):
            for ri, r in enumerate(rings):
                ag[ri, 0, g].wait_recv()
                ag[ri, 1, g] = start(ag_rdma(r, 1, g))
                r["p"][0, rows(g), :] = f(r, 1, g)
                rs[ri, 0, g] = start(rs_rdma(r, 0, g))

        for g in range(SEG):
            for ri, r in enumerate(rings):
                ag[ri, 1, g].wait_recv()
                ag[ri, 2, g] = start(ag_rdma(r, 2, g))
                r["p"][1, rows(g), :] = f(r, 2, g)

        for g in range(SEG):
            for ri, r in enumerate(rings):
                rs[ri, 0, g].wait_recv()
                r["p"][1, rows(g), :] = r["p"][1, rows(g), :] + r["rs"][0, rows(g), :]
                rs[ri, 1, g] = start(rs_rdma(r, 1, g))

        for g in range(SEG):
            for ri, r in enumerate(rings):
                ag[ri, 2, g].wait_recv()
                r["p"][2, rows(g), :] = f(r, 3, g)

        for g in range(SEG):
            for ri, r in enumerate(rings):
                rs[ri, 1, g].wait_recv()
                r["p"][2, rows(g), :] = r["p"][2, rows(g), :] + r["rs"][1, rows(g), :]
                rs[ri, 2, g] = start(rs_rdma(r, 2, g))

        for g in range(SEG):
            for ri, r in enumerate(rings):
                rs[ri, 2, g].wait_recv()
                o = slice(r["off"] + g * sr, r["off"] + (g + 1) * sr)
                out_ref[o, :] = r["p"][3, rows(g), :] + r["rs"][2, rows(g), :]

        for desc in sends:
            desc.wait_send()

    half = (N_DEV, mh, d)
    rs_shape = (N_DEV - 1, mh, d)
    sem2 = pltpu.SemaphoreType.DMA((N_DEV - 1, SEG))
    return pl.pallas_call(
        body,
        out_shape=jax.ShapeDtypeStruct((m_per, d), jnp.float32),
        in_specs=[pl.BlockSpec(memory_space=pltpu.VMEM)] * 3,
        out_specs=pl.BlockSpec(memory_space=pltpu.VMEM),
        scratch_shapes=[
            pltpu.VMEM(half, jnp.float32),
            pltpu.VMEM(half, jnp.float32),
            pltpu.VMEM(rs_shape, jnp.float32),
            pltpu.VMEM(half, jnp.float32),
            pltpu.VMEM(half, jnp.float32),
            pltpu.VMEM(rs_shape, jnp.float32),
            sem2, sem2, sem2, sem2,
            sem2, sem2, sem2, sem2,
        ],
        compiler_params=pltpu.CompilerParams(collective_id=0),
    )(x, W1, W2)
